# Initial kernel scaffold; baseline (speedup 1.0000x reference)
import jax
import jax.numpy as jnp
from jax.experimental import pallas as pl


def kernel(prediction, target):
    raise NotImplementedError("write your pallas kernel here")


# speedup vs baseline: 76.4083x; 76.4083x over previous
import functools

import jax
import jax.numpy as jnp
from jax import lax
from jax.experimental import pallas as pl
from jax.experimental.pallas import tpu as pltpu
from jax.experimental.pallas import tpu_sc as plsc

N = 262144
NUM_CORES = 2
NUM_SUBCORES = 16
LANES = 16
NW = NUM_CORES * NUM_SUBCORES
CHUNK = N // NW
STEPS = CHUNK // LANES


def _body(pred_hbm, targ_hbm, sum_hbm, cnt_hbm, pred_v, targ_v, out_s, out_c):
    cid = lax.axis_index("c")
    sid = lax.axis_index("s")
    wid = sid * NUM_CORES + cid
    base = wid * CHUNK
    pltpu.sync_copy(pred_hbm.at[pl.ds(base, CHUNK)], pred_v)
    pltpu.sync_copy(targ_hbm.at[pl.ds(base, CHUNK)], targ_v)

    def step(i, carry):
        s, c = carry
        p = pred_v[pl.ds(i * LANES, LANES)]
        t = targ_v[pl.ds(i * LANES, LANES)]
        d = p - t
        s = s + jnp.abs(d)
        ok = (d > -0.5) & ((d <= 0.5) | (t == 99.0))
        c = c + jnp.where(ok, jnp.full((LANES,), 1, jnp.int32),
                          jnp.full((LANES,), 0, jnp.int32))
        return s, c

    s, c = lax.fori_loop(
        0, STEPS, step,
        (jnp.zeros((LANES,), jnp.float32), jnp.zeros((LANES,), jnp.int32)),
    )
    out_s[...] = s
    out_c[...] = c
    pltpu.sync_copy(out_s, sum_hbm.at[pl.ds(wid * LANES, LANES)])
    pltpu.sync_copy(out_c, cnt_hbm.at[pl.ds(wid * LANES, LANES)])


@jax.jit
def _focus_metrics(pred_flat, target):
    sums, cnts = pl.kernel(
        _body,
        out_type=(
            jax.ShapeDtypeStruct((NW * LANES,), jnp.float32),
            jax.ShapeDtypeStruct((NW * LANES,), jnp.int32),
        ),
        mesh=plsc.VectorSubcoreMesh(
            core_axis_name="c", subcore_axis_name="s",
            num_cores=NUM_CORES, num_subcores=NUM_SUBCORES,
        ),
        scratch_types=[
            pltpu.VMEM((CHUNK,), jnp.float32),
            pltpu.VMEM((CHUNK,), jnp.float32),
            pltpu.VMEM((LANES,), jnp.float32),
            pltpu.VMEM((LANES,), jnp.int32),
        ],
    )(pred_flat, target)
    l1 = jnp.sum(sums) / jnp.float32(N)
    correct = jnp.sum(cnts)
    total = jnp.array(N, dtype=jnp.int32)
    return l1, correct, total


def kernel(prediction, target):
    return _focus_metrics(prediction.reshape(-1), target)


# speedup vs baseline: 78.4918x; 1.0273x over previous
import functools

import jax
import jax.numpy as jnp
from jax import lax
from jax.experimental import pallas as pl
from jax.experimental.pallas import tpu as pltpu
from jax.experimental.pallas import tpu_sc as plsc

N = 262144
NUM_CORES = 2
NUM_SUBCORES = 16
LANES = 16
NW = NUM_CORES * NUM_SUBCORES
CHUNK = N // NW
STEPS = CHUNK // LANES
UNROLL = 8


def _body(pred_hbm, targ_hbm, sum_hbm, cnt_hbm, pred_v, targ_v, out_s, out_c):
    cid = lax.axis_index("c")
    sid = lax.axis_index("s")
    wid = sid * NUM_CORES + cid
    base = wid * CHUNK
    pltpu.sync_copy(pred_hbm.at[pl.ds(base, CHUNK)], pred_v)
    pltpu.sync_copy(targ_hbm.at[pl.ds(base, CHUNK)], targ_v)

    ones = jnp.full((LANES,), 1, jnp.int32)
    zeros = jnp.full((LANES,), 0, jnp.int32)

    def step(k, carry):
        ss, cs = carry
        base = k * (UNROLL * LANES)
        new_ss, new_cs = [], []
        for u in range(UNROLL):
            p = pred_v[pl.ds(base + u * LANES, LANES)]
            t = targ_v[pl.ds(base + u * LANES, LANES)]
            d = p - t
            ok = (d > -0.5) & ((d <= 0.5) | (t == 99.0))
            new_ss.append(ss[u] + jnp.abs(d))
            new_cs.append(cs[u] + jnp.where(ok, ones, zeros))
        return tuple(new_ss), tuple(new_cs)

    zs = jnp.zeros((LANES,), jnp.float32)
    ss, cs = lax.fori_loop(
        0, STEPS // UNROLL, step,
        (tuple(zs for _ in range(UNROLL)), tuple(zeros for _ in range(UNROLL))),
    )
    s = ss[0]
    c = cs[0]
    for u in range(1, UNROLL):
        s = s + ss[u]
        c = c + cs[u]
    out_s[...] = s
    out_c[...] = c
    pltpu.sync_copy(out_s, sum_hbm.at[pl.ds(wid * LANES, LANES)])
    pltpu.sync_copy(out_c, cnt_hbm.at[pl.ds(wid * LANES, LANES)])


@jax.jit
def _focus_metrics(pred_flat, target):
    sums, cnts = pl.kernel(
        _body,
        out_type=(
            jax.ShapeDtypeStruct((NW * LANES,), jnp.float32),
            jax.ShapeDtypeStruct((NW * LANES,), jnp.int32),
        ),
        mesh=plsc.VectorSubcoreMesh(
            core_axis_name="c", subcore_axis_name="s",
            num_cores=NUM_CORES, num_subcores=NUM_SUBCORES,
        ),
        scratch_types=[
            pltpu.VMEM((CHUNK,), jnp.float32),
            pltpu.VMEM((CHUNK,), jnp.float32),
            pltpu.VMEM((LANES,), jnp.float32),
            pltpu.VMEM((LANES,), jnp.int32),
        ],
    )(pred_flat, target)
    l1 = jnp.sum(sums) / jnp.float32(N)
    correct = jnp.sum(cnts)
    total = jnp.array(N, dtype=jnp.int32)
    return l1, correct, total


def kernel(prediction, target):
    return _focus_metrics(prediction.reshape(-1), target)
